# baseline (speedup 1.0000x reference)
import jax
import jax.numpy as jnp
from jax.experimental import pallas as pl
from jax.experimental.pallas import tpu as pltpu

_EPS = 1e-5


def _setnorm_kernel(x_ref, w_ref, b_ref, o_ref):
    N, F = x_ref.shape[1], x_ref.shape[2]
    n = N * F
    R = 64
    C = N // R
    acc1 = jnp.zeros((R, F), jnp.float32)
    acc2 = jnp.zeros((R, F), jnp.float32)
    for i in range(C):
        xs = x_ref[0, i * R:(i + 1) * R, :]
        acc1 = acc1 + xs
        acc2 = acc2 + xs * xs
    s1 = jnp.sum(acc1, keepdims=True)
    s2 = jnp.sum(acc2, keepdims=True)
    mean = s1 * (1.0 / n)
    var = s2 * (1.0 / n) - mean * mean
    inv = jax.lax.rsqrt(var + _EPS)
    scale = w_ref[...] * inv
    shift = b_ref[...] - mean * scale
    CN = 8
    step = N // CN
    for i in range(CN):
        lo, hi = i * step, (i + 1) * step
        o_ref[0, lo:hi, :] = x_ref[0, lo:hi, :] * scale + shift


@jax.jit
def kernel(x, weights, biases):
    B, N, F = x.shape
    w2 = weights.reshape(1, F)
    b2 = biases.reshape(1, F)
    return pl.pallas_call(
        _setnorm_kernel,
        grid=(B,),
        in_specs=[
            pl.BlockSpec((1, N, F), lambda b: (b, 0, 0)),
            pl.BlockSpec((1, F), lambda b: (0, 0)),
            pl.BlockSpec((1, F), lambda b: (0, 0)),
        ],
        out_specs=pl.BlockSpec((1, N, F), lambda b: (b, 0, 0)),
        out_shape=jax.ShapeDtypeStruct((B, N, F), x.dtype),
        compiler_params=pltpu.CompilerParams(
            dimension_semantics=("parallel",),
            vmem_limit_bytes=52 * 1024 * 1024,
        ),
    )(x, w2, b2)


# speedup vs baseline: 1.0020x; 1.0020x over previous
import jax
import jax.numpy as jnp
from jax.experimental import pallas as pl
from jax.experimental.pallas import tpu as pltpu

_EPS = 1e-5


def _setnorm_kernel(x_ref, w_ref, b_ref, o_ref):
    N, F = x_ref.shape[1], x_ref.shape[2]
    n = N * F
    R = 64
    C = N // R
    acc1 = jnp.zeros((R, F), jnp.float32)
    acc2 = jnp.zeros((R, F), jnp.float32)
    for i in range(C):
        xs = x_ref[0, i * R:(i + 1) * R, :]
        acc1 = acc1 + xs
        acc2 = acc2 + xs * xs
    s1 = jnp.sum(acc1, keepdims=True)
    s2 = jnp.sum(acc2, keepdims=True)
    mean = s1 * (1.0 / n)
    var = s2 * (1.0 / n) - mean * mean
    inv = jax.lax.rsqrt(var + _EPS)
    scale = w_ref[...] * inv
    shift = b_ref[...] - mean * scale
    CN = 16
    step = N // CN
    for i in range(CN):
        lo, hi = i * step, (i + 1) * step
        o_ref[0, lo:hi, :] = x_ref[0, lo:hi, :] * scale + shift


@jax.jit
def kernel(x, weights, biases):
    B, N, F = x.shape
    w2 = weights.reshape(1, F)
    b2 = biases.reshape(1, F)
    return pl.pallas_call(
        _setnorm_kernel,
        grid=(B,),
        in_specs=[
            pl.BlockSpec((1, N, F), lambda b: (b, 0, 0)),
            pl.BlockSpec((1, F), lambda b: (0, 0)),
            pl.BlockSpec((1, F), lambda b: (0, 0)),
        ],
        out_specs=pl.BlockSpec((1, N, F), lambda b: (b, 0, 0)),
        out_shape=jax.ShapeDtypeStruct((B, N, F), x.dtype),
        compiler_params=pltpu.CompilerParams(
            dimension_semantics=("parallel",),
            vmem_limit_bytes=52 * 1024 * 1024,
        ),
    )(x, w2, b2)
